# baseline (speedup 1.0000x reference)
import jax
import jax.numpy as jnp
from jax import lax
from jax.experimental import pallas as pl
from jax.experimental.pallas import tpu as pltpu
from jax.experimental.pallas import tpu_sc as plsc

_K = 65536
_DIM = 128
_B = 1024

_NC = 2
_NS = 16
_NW = _NC * _NS
_SLAB = _K // _NW
_CROWS = 256
_NCH = _SLAB // _CROWS
_KCH = _B // _CROWS


def _copy_slab(q_hbm, out_hbm, base, bufs, in_sems, out_sems):
    ins = [None] * _NCH
    outs = [None] * _NCH

    def fire_in(i):
        b = i % 2
        return pltpu.async_copy(
            q_hbm.at[pl.ds(base + i * _CROWS, _CROWS), :], bufs[b],
            in_sems.at[b])

    ins[0] = fire_in(0)
    for i in range(_NCH):
        b = i % 2
        if i + 1 < _NCH:
            if i - 1 >= 0:
                outs[i - 1].wait()
            ins[i + 1] = fire_in(i + 1)
        ins[i].wait()
        outs[i] = pltpu.async_copy(
            bufs[b], out_hbm.at[pl.ds(base + i * _CROWS, _CROWS), :],
            out_sems.at[b])
    if _NCH >= 2:
        outs[_NCH - 2].wait()
    outs[_NCH - 1].wait()


def _enqueue_body(ptr1_hbm, ptr2_hbm, keys1_hbm, keys2_hbm, q1_hbm, q2_hbm,
                  out1_hbm, out2_hbm, p1_v, p2_v, buf0, buf1, in_sems,
                  out_sems):
    wid = lax.axis_index("s") * _NC + lax.axis_index("c")
    base = wid * _SLAB
    bufs = (buf0, buf1)

    pltpu.sync_copy(ptr1_hbm, p1_v)
    pltpu.sync_copy(ptr2_hbm, p2_v)
    p1 = pl.multiple_of(p1_v[...][0], _B)
    p2 = pl.multiple_of(p2_v[...][0], _B)

    _copy_slab(q1_hbm, out1_hbm, base, bufs, in_sems, out_sems)
    _copy_slab(q2_hbm, out2_hbm, base, bufs, in_sems, out_sems)

    for p, keys_hbm, out_hbm in ((p1, keys1_hbm, out1_hbm),
                                 (p2, keys2_hbm, out2_hbm)):
        @pl.when(jnp.logical_and(p >= base, p < base + _SLAB))
        def _():
            for kb in range(_KCH):
                pltpu.sync_copy(keys_hbm.at[pl.ds(kb * _CROWS, _CROWS), :],
                                buf0)
                pltpu.sync_copy(buf0,
                                out_hbm.at[pl.ds(p + kb * _CROWS, _CROWS), :])


def kernel(keys_1, keys_2, queue_1, queue_2, queue_1_ptr, queue_2_ptr):
    mesh = plsc.VectorSubcoreMesh(core_axis_name="c", subcore_axis_name="s")
    out_t = (jax.ShapeDtypeStruct((_K, _DIM), jnp.float32),
             jax.ShapeDtypeStruct((_K, _DIM), jnp.float32))
    run = pl.kernel(
        _enqueue_body,
        out_type=out_t,
        mesh=mesh,
        scratch_types=[
            pltpu.VMEM((16,), jnp.int32),
            pltpu.VMEM((16,), jnp.int32),
            pltpu.VMEM((_CROWS, _DIM), jnp.float32),
            pltpu.VMEM((_CROWS, _DIM), jnp.float32),
            pltpu.SemaphoreType.DMA((2,)),
            pltpu.SemaphoreType.DMA((2,)),
        ],
    )
    ptr1_v = jnp.broadcast_to(queue_1_ptr, (16,)).astype(jnp.int32)
    ptr2_v = jnp.broadcast_to(queue_2_ptr, (16,)).astype(jnp.int32)
    q1_new, q2_new = run(ptr1_v, ptr2_v, keys_1, keys_2, queue_1, queue_2)
    ptr1_new = ((queue_1_ptr + _B) % _K).astype(jnp.int32)
    ptr2_new = ((queue_2_ptr + _B) % _K).astype(jnp.int32)
    return q1_new, q2_new, ptr1_new, ptr2_new


# speedup vs baseline: 1.1035x; 1.1035x over previous
import jax
import jax.numpy as jnp
from jax import lax
from jax.experimental import pallas as pl
from jax.experimental.pallas import tpu as pltpu
from jax.experimental.pallas import tpu_sc as plsc

_K = 65536
_DIM = 128
_B = 1024

_NC = 2
_NS = 16
_NW = _NC * _NS
_SLAB = _K // _NW
_CROWS = 128
_NB = 6
_PRIME = 4


def _pipe_copy(pairs, bufs, in_sems, out_sems):
    n = len(pairs)
    prime = min(_PRIME, n)
    ins = [None] * n
    outs = [None] * n

    def fire_in(j):
        return pltpu.async_copy(pairs[j][0], bufs[j % _NB],
                                in_sems.at[j % _NB])

    for j in range(prime):
        ins[j] = fire_in(j)
    drained = set()
    for j in range(n):
        nxt = j + prime
        if nxt < n:
            prev = nxt - _NB
            if prev >= 0:
                outs[prev].wait()
                drained.add(prev)
            ins[nxt] = fire_in(nxt)
        ins[j].wait()
        outs[j] = pltpu.async_copy(bufs[j % _NB], pairs[j][1],
                                   out_sems.at[j % _NB])
    for j in range(n):
        if j not in drained:
            outs[j].wait()


def _chunks(src, dst, rows, src_base, dst_base):
    return [(src.at[pl.ds(src_base + r, _CROWS), :],
             dst.at[pl.ds(dst_base + r, _CROWS), :])
            for r in range(0, rows, _CROWS)]


def _enqueue_body(ptr1_hbm, ptr2_hbm, keys1_hbm, keys2_hbm, q1_hbm, q2_hbm,
                  out1_hbm, out2_hbm, p1_v, p2_v, bufs, in_sems, out_sems):
    wid = lax.axis_index("s") * _NC + lax.axis_index("c")
    base = wid * _SLAB

    pltpu.sync_copy(ptr1_hbm, p1_v)
    pltpu.sync_copy(ptr2_hbm, p2_v)
    p1 = pl.multiple_of(p1_v[...][0], _B)
    p2 = pl.multiple_of(p2_v[...][0], _B)

    slab_pairs = (_chunks(q1_hbm, out1_hbm, _SLAB, base, base)
                  + _chunks(q2_hbm, out2_hbm, _SLAB, base, base))
    _pipe_copy(slab_pairs, bufs, in_sems, out_sems)

    for p, keys_hbm, out_hbm in ((p1, keys1_hbm, out1_hbm),
                                 (p2, keys2_hbm, out2_hbm)):
        @pl.when(jnp.logical_and(p >= base, p < base + _SLAB))
        def _():
            _pipe_copy(_chunks(keys_hbm, out_hbm, _B, 0, p),
                       bufs, in_sems, out_sems)


def kernel(keys_1, keys_2, queue_1, queue_2, queue_1_ptr, queue_2_ptr):
    mesh = plsc.VectorSubcoreMesh(core_axis_name="c", subcore_axis_name="s")
    out_t = (jax.ShapeDtypeStruct((_K, _DIM), jnp.float32),
             jax.ShapeDtypeStruct((_K, _DIM), jnp.float32))
    run = pl.kernel(
        _enqueue_body,
        out_type=out_t,
        mesh=mesh,
        scratch_types=[
            pltpu.VMEM((16,), jnp.int32),
            pltpu.VMEM((16,), jnp.int32),
            [pltpu.VMEM((_CROWS, _DIM), jnp.float32) for _ in range(_NB)],
            pltpu.SemaphoreType.DMA((_NB,)),
            pltpu.SemaphoreType.DMA((_NB,)),
        ],
    )
    ptr1_v = jnp.broadcast_to(queue_1_ptr, (16,)).astype(jnp.int32)
    ptr2_v = jnp.broadcast_to(queue_2_ptr, (16,)).astype(jnp.int32)
    q1_new, q2_new = run(ptr1_v, ptr2_v, keys_1, keys_2, queue_1, queue_2)
    ptr1_new = ((queue_1_ptr + _B) % _K).astype(jnp.int32)
    ptr2_new = ((queue_2_ptr + _B) % _K).astype(jnp.int32)
    return q1_new, q2_new, ptr1_new, ptr2_new


# speedup vs baseline: 1.2257x; 1.1107x over previous
import jax
import jax.numpy as jnp
from jax import lax
from jax.experimental import pallas as pl
from jax.experimental.pallas import tpu as pltpu
from jax.experimental.pallas import tpu_sc as plsc

_K = 65536
_DIM = 128
_B = 1024

_NC = 2
_NS = 16
_NW = _NC * _NS
_SLAB = _K // _NW
_CROWS = 128
_NB = 6
_PRIME = 4


def _pipe_copy(pairs, bufs, in_sems, out_sems):
    n = len(pairs)
    prime = min(_PRIME, n)
    ins = [None] * n
    outs = [None] * n

    def fire_in(j):
        return pltpu.async_copy(pairs[j][0], bufs[j % _NB],
                                in_sems.at[j % _NB])

    for j in range(prime):
        ins[j] = fire_in(j)
    drained = set()
    for j in range(n):
        nxt = j + prime
        if nxt < n:
            prev = nxt - _NB
            if prev >= 0:
                outs[prev].wait()
                drained.add(prev)
            ins[nxt] = fire_in(nxt)
        ins[j].wait()
        outs[j] = pltpu.async_copy(bufs[j % _NB], pairs[j][1],
                                   out_sems.at[j % _NB])
    for j in range(n):
        if j not in drained:
            outs[j].wait()


def _chunks(src, dst, rows, src_base, dst_base):
    return [(src.at[pl.ds(src_base + r, _CROWS), :],
             dst.at[pl.ds(dst_base + r, _CROWS), :])
            for r in range(0, rows, _CROWS)]


def _enqueue_body(ptr1_hbm, ptr2_hbm, keys1_hbm, keys2_hbm, q1_hbm, q2_hbm,
                  out1_hbm, out2_hbm, p1_v, p2_v, bufs, in_sems, out_sems):
    sid = lax.axis_index("s")
    wid = sid * _NC + lax.axis_index("c")
    base = wid * _SLAB
    bufs = [b.at[sid] for b in bufs]

    pltpu.sync_copy(ptr1_hbm, p1_v)
    pltpu.sync_copy(ptr2_hbm, p2_v)
    p1 = pl.multiple_of(p1_v[...][0], _B)
    p2 = pl.multiple_of(p2_v[...][0], _B)

    slab_pairs = (_chunks(q1_hbm, out1_hbm, _SLAB, base, base)
                  + _chunks(q2_hbm, out2_hbm, _SLAB, base, base))
    _pipe_copy(slab_pairs, bufs, in_sems, out_sems)

    for p, keys_hbm, out_hbm in ((p1, keys1_hbm, out1_hbm),
                                 (p2, keys2_hbm, out2_hbm)):
        @pl.when(jnp.logical_and(p >= base, p < base + _SLAB))
        def _():
            _pipe_copy(_chunks(keys_hbm, out_hbm, _B, 0, p),
                       bufs, in_sems, out_sems)


def kernel(keys_1, keys_2, queue_1, queue_2, queue_1_ptr, queue_2_ptr):
    mesh = plsc.VectorSubcoreMesh(core_axis_name="c", subcore_axis_name="s")
    out_t = (jax.ShapeDtypeStruct((_K, _DIM), jnp.float32),
             jax.ShapeDtypeStruct((_K, _DIM), jnp.float32))
    run = pl.kernel(
        _enqueue_body,
        out_type=out_t,
        mesh=mesh,
        scratch_types=[
            pltpu.VMEM((16,), jnp.int32),
            pltpu.VMEM((16,), jnp.int32),
            [pltpu.VMEM_SHARED((_NS, _CROWS, _DIM), jnp.float32)
             for _ in range(_NB)],
            pltpu.SemaphoreType.DMA((_NB,)),
            pltpu.SemaphoreType.DMA((_NB,)),
        ],
    )
    ptr1_v = jnp.broadcast_to(queue_1_ptr, (16,)).astype(jnp.int32)
    ptr2_v = jnp.broadcast_to(queue_2_ptr, (16,)).astype(jnp.int32)
    q1_new, q2_new = run(ptr1_v, ptr2_v, keys_1, keys_2, queue_1, queue_2)
    ptr1_new = ((queue_1_ptr + _B) % _K).astype(jnp.int32)
    ptr2_new = ((queue_2_ptr + _B) % _K).astype(jnp.int32)
    return q1_new, q2_new, ptr1_new, ptr2_new
